# baseline (speedup 1.0000x reference)
import functools

import jax
import jax.numpy as jnp
from jax import lax
from jax.experimental import pallas as pl
from jax.experimental.pallas import tpu as pltpu
from jax.experimental.pallas import tpu_sc as plsc

V = 100000
B = 1024
L = 20
PAD = 1

NC = 2
NS = 16
LANES = 16
NW = NC * NS
ROWS_PER_W = B // NW
TOK_PER_W = ROWS_PER_W * L
CHUNKS = TOK_PER_W // 128


def _body(ids_hbm, w_hbm, out_hbm, zbuf, ids_v, vals_v, gidx_v,
          sem_z, sem_g, sem_s):
    cid = lax.axis_index("c")
    sid = lax.axis_index("s")
    wid = sid * NC + cid
    base_row = wid * ROWS_PER_W

    def zloop(i, carry):
        base = i * (LANES * 10)
        for k in range(10):
            off = pl.multiple_of(base + k * LANES, LANES)
            zbuf[pl.ds(off, LANES)] = jnp.zeros((LANES,), jnp.float32)
        return carry
    lax.fori_loop(0, V // (LANES * 10), zloop, 0)

    zcopies = []
    for j in range(ROWS_PER_W):
        row = base_row + j
        cp = pltpu.make_async_copy(
            zbuf, out_hbm.at[pl.ds(row * V, V)], sem_z)
        cp.start()
        zcopies.append(cp)

    pltpu.sync_copy(ids_hbm.at[wid], ids_v)

    gcopies = []
    for j in range(CHUNKS):
        cp = pltpu.make_async_copy(
            w_hbm.at[ids_v.at[j]], vals_v.at[j], sem_g)
        cp.start()
        gcopies.append(cp)
    for cp in gcopies:
        cp.wait()

    for c in range(TOK_PER_W // LANES):
        j, o = divmod(c, 128 // LANES)
        sl = pl.ds(o * LANES, LANES)
        col = ids_v[j, sl]
        lp = c * LANES + lax.iota(jnp.int32, LANES)
        row = base_row + lax.div(lp, jnp.int32(L))
        gidx_v[j, sl] = row * V + col
        vals_v[j, sl] = jnp.where(col == PAD, 0.0, vals_v[j, sl])

    for cp in zcopies:
        cp.wait()

    scopies = []
    for j in range(CHUNKS):
        cp = pltpu.make_async_copy(
            vals_v.at[j], out_hbm.at[gidx_v.at[j]], sem_s)
        cp.start()
        scopies.append(cp)
    for cp in scopies:
        cp.wait()


@jax.jit
def kernel(input_ids, weights):
    ids3 = input_ids.astype(jnp.int32).reshape(NW, CHUNKS, 128)
    mesh = plsc.VectorSubcoreMesh(
        core_axis_name="c", subcore_axis_name="s",
        num_cores=NC, num_subcores=NS)
    out_flat = pl.kernel(
        _body,
        out_type=jax.ShapeDtypeStruct((B * V,), jnp.float32),
        mesh=mesh,
        scratch_types=[
            pltpu.VMEM((V,), jnp.float32),
            pltpu.VMEM((CHUNKS, 128), jnp.int32),
            pltpu.VMEM((CHUNKS, 128), jnp.float32),
            pltpu.VMEM((CHUNKS, 128), jnp.int32),
            pltpu.SemaphoreType.DMA,
            pltpu.SemaphoreType.DMA,
            pltpu.SemaphoreType.DMA,
        ],
    )(ids3, weights)
    return out_flat.reshape(B, V)


# speedup vs baseline: 1.0014x; 1.0014x over previous
import jax
import jax.numpy as jnp
from jax import lax
from jax.experimental import pallas as pl
from jax.experimental.pallas import tpu as pltpu
from jax.experimental.pallas import tpu_sc as plsc

V = 100000
B = 1024
L = 20
PAD = 1

NC = 2
NS = 16
LANES = 16
NW = NC * NS
ROWS_PER_W = B // NW
TOK_PER_W = ROWS_PER_W * L
CHUNKS = TOK_PER_W // 128

NBLK = 50
ZBLK = B * V // NBLK


def _zero_body(o_ref):
    o_ref[...] = jnp.zeros_like(o_ref)


def _scatter_body(ids_hbm, w_hbm, out_hbm, ids_v, vals_v, gidx_v,
                  sem_g, sem_s):
    cid = lax.axis_index("c")
    sid = lax.axis_index("s")
    wid = sid * NC + cid
    base_row = wid * ROWS_PER_W

    pltpu.sync_copy(ids_hbm.at[wid], ids_v)

    gcopies = []
    for j in range(CHUNKS):
        cp = pltpu.make_async_copy(
            w_hbm.at[ids_v.at[j]], vals_v.at[j], sem_g)
        cp.start()
        gcopies.append(cp)
    for cp in gcopies:
        cp.wait()

    for c in range(TOK_PER_W // LANES):
        j, o = divmod(c, 128 // LANES)
        sl = pl.ds(o * LANES, LANES)
        col = ids_v[j, sl]
        lp = c * LANES + lax.iota(jnp.int32, LANES)
        row = base_row + lax.div(lp, jnp.int32(L))
        gidx_v[j, sl] = row * V + col
        vals_v[j, sl] = jnp.where(col == PAD, 0.0, vals_v[j, sl])

    scopies = []
    for j in range(CHUNKS):
        cp = pltpu.make_async_copy(
            vals_v.at[j], out_hbm.at[gidx_v.at[j]], sem_s)
        cp.start()
        scopies.append(cp)
    for cp in scopies:
        cp.wait()


@jax.jit
def kernel(input_ids, weights):
    ids3 = input_ids.astype(jnp.int32).reshape(NW, CHUNKS, 128)

    zeros = pl.pallas_call(
        _zero_body,
        out_shape=jax.ShapeDtypeStruct((B * V,), jnp.float32),
        grid=(NBLK,),
        out_specs=pl.BlockSpec((ZBLK,), lambda i: (i,)),
    )()

    out_ref = jax.new_ref(zeros)
    mesh = plsc.VectorSubcoreMesh(
        core_axis_name="c", subcore_axis_name="s",
        num_cores=NC, num_subcores=NS)
    pl.kernel(
        _scatter_body,
        out_type=(),
        mesh=mesh,
        scratch_types=[
            pltpu.VMEM((CHUNKS, 128), jnp.int32),
            pltpu.VMEM((CHUNKS, 128), jnp.float32),
            pltpu.VMEM((CHUNKS, 128), jnp.int32),
            pltpu.SemaphoreType.DMA,
            pltpu.SemaphoreType.DMA,
        ],
    )(ids3, weights, out_ref)
    return out_ref[...].reshape(B, V)
